# baseline (speedup 1.0000x reference)
import functools

import jax
import jax.numpy as jnp
from jax import lax
from jax.experimental import pallas as pl
from jax.experimental.pallas import tpu as pltpu
from jax.experimental.pallas import tpu_sc as plsc

_L = 16


@jax.jit
def _gather(node, table):
    B, = node.shape
    V, D = table.shape
    info = plsc.get_sparse_core_info()
    NC, NS = info.num_cores, info.num_subcores
    NW = NC * NS
    b_per_w = B // NW
    CH = 128
    n_ch = b_per_w // CH
    mesh = plsc.VectorSubcoreMesh(core_axis_name="c", subcore_axis_name="s")

    @functools.partial(
        pl.kernel,
        mesh=mesh,
        out_type=jax.ShapeDtypeStruct((B, D), jnp.float32),
        scratch_types=[
            pltpu.VMEM((b_per_w,), jnp.int32),
            pltpu.VMEM((b_per_w // 64, 128), jnp.int32),
            pltpu.VMEM((2 * b_per_w, D), jnp.float32),
            pltpu.SemaphoreType.DMA,
        ],
        compiler_params=pltpu.CompilerParams(
            use_tc_tiling_on_sc=False,
            needs_layout_passes=False,
        ),
    )
    def k(node_hbm, table_hbm, out_hbm, nd_v, ix_v, rows_v, sem):
        wid = lax.axis_index("s") * NC + lax.axis_index("c")
        base = wid * b_per_w
        pltpu.sync_copy(node_hbm.at[pl.ds(base, b_per_w)], nd_v)

        zero = jnp.zeros((_L,), jnp.int32)
        for r in range(b_per_w // 64):
            for t in range(128 // _L):
                ix_v[r, pl.ds(t * _L, _L)] = zero

        lane2 = lax.iota(jnp.int32, _L) << 1
        for t in range(b_per_w // _L):
            v = nd_v[pl.ds(t * _L, _L)] << 1
            p = lane2 + (2 * _L * t)
            plsc.store_scatter(ix_v, [p >> 7, p & 127], v)

        copies = [
            pltpu.async_copy(
                table_hbm.at[ix_v.at[j]],
                rows_v.at[pl.ds(j * 128, 128)],
                sem,
            )
            for j in range(b_per_w // 64)
        ]
        for c in copies:
            c.wait()

        for j in range(b_per_w // 64):
            pltpu.sync_copy(
                rows_v.at[pl.ds(j * 128, 64)],
                out_hbm.at[pl.ds(base + j * 64, 64)],
            )

    return k(node, table)


def kernel(node, table):
    return _gather(node.astype(jnp.int32), table)


# speedup vs baseline: 1.9204x; 1.9204x over previous
import functools

import jax
import jax.numpy as jnp
from jax import lax
from jax.experimental import pallas as pl
from jax.experimental.pallas import tpu as pltpu
from jax.experimental.pallas import tpu_sc as plsc

_L = 16
_RPS = 32


@jax.jit
def _gather(node, table):
    B, = node.shape
    V, D = table.shape
    table_sr = table.reshape(V // _RPS, _RPS * D)
    info = plsc.get_sparse_core_info()
    NC, NS = info.num_cores, info.num_subcores
    NW = NC * NS
    b_per_w = B // NW
    CH = 128
    n_ch = b_per_w // CH
    mesh = plsc.VectorSubcoreMesh(core_axis_name="c", subcore_axis_name="s")

    @functools.partial(
        pl.kernel,
        mesh=mesh,
        out_type=jax.ShapeDtypeStruct((B, D), jnp.float32),
        scratch_types=[
            pltpu.VMEM((b_per_w,), jnp.int32),
            pltpu.VMEM((n_ch, CH), jnp.int32),
            pltpu.VMEM((2, CH, _RPS * D), jnp.float32),
            pltpu.VMEM((b_per_w, D), jnp.float32),
            pltpu.SemaphoreType.DMA,
            pltpu.SemaphoreType.DMA,
        ],
        compiler_params=pltpu.CompilerParams(needs_layout_passes=False),
    )
    def k(node_hbm, table_hbm, out_hbm, nd_v, sr_v, rows_v, out_v, sem0, sem1):
        wid = lax.axis_index("s") * NC + lax.axis_index("c")
        base = wid * b_per_w
        sems = [sem0, sem1]
        pltpu.sync_copy(node_hbm.at[pl.ds(base, b_per_w)], nd_v)

        for j in range(n_ch):
            for t in range(CH // _L):
                nd = nd_v[pl.ds(j * CH + t * _L, _L)]
                sr_v[j, pl.ds(t * _L, _L)] = nd >> 5

        def start(j):
            return pltpu.async_copy(
                table_hbm.at[sr_v.at[j]], rows_v.at[j % 2], sems[j % 2])

        lane = lax.iota(jnp.int32, _L)
        bid0 = lane >> 2
        col0 = lane & 3
        copies = [start(0)]
        for j in range(n_ch):
            if j + 1 < n_ch:
                copies.append(start(j + 1))
            copies[j].wait()
            buf = rows_v.at[j % 2]
            for v in range(CH * D // _L):
                bid = bid0 + (v * _L // D)
                nd = plsc.load_gather(nd_v, [j * CH + bid])
                col = ((nd & (_RPS - 1)) << 2) + col0
                vals = plsc.load_gather(buf, [bid, col])
                plsc.store_scatter(out_v, [j * CH + bid, col0], vals)

        pltpu.sync_copy(out_v, out_hbm.at[pl.ds(base, b_per_w)])

    return k(node, table_sr)


def kernel(node, table):
    return _gather(node.astype(jnp.int32), table)


# speedup vs baseline: 4.7310x; 2.4635x over previous
import functools

import jax
import jax.numpy as jnp
from jax import lax
from jax.experimental import pallas as pl
from jax.experimental.pallas import tpu as pltpu
from jax.experimental.pallas import tpu_sc as plsc

_L = 16


@jax.jit
def _gather(node, table):
    B, = node.shape
    V, D = table.shape
    cols = lax.optimization_barrier(tuple(table[:, c] for c in range(D)))
    info = plsc.get_sparse_core_info()
    NC, NS = info.num_cores, info.num_subcores
    NW = NC * NS
    b_per_w = B // NW
    CH = 128
    n_ch = b_per_w // CH
    mesh = plsc.VectorSubcoreMesh(core_axis_name="c", subcore_axis_name="s")

    @functools.partial(
        pl.kernel,
        mesh=mesh,
        out_type=jax.ShapeDtypeStruct((B, D), jnp.float32),
        scratch_types=[
            pltpu.VMEM((n_ch, CH), jnp.int32),
            pltpu.VMEM((D, b_per_w), jnp.float32),
            pltpu.VMEM((b_per_w, D), jnp.float32),
            pltpu.SemaphoreType.DMA,
        ],
        compiler_params=pltpu.CompilerParams(needs_layout_passes=False),
    )
    def k(node_hbm, c0, c1, c2, c3, out_hbm, nd_v, colbuf, out_v, sem):
        wid = lax.axis_index("s") * NC + lax.axis_index("c")
        base = wid * b_per_w
        pltpu.sync_copy(node_hbm.at[pl.ds(wid * n_ch, n_ch)], nd_v)

        copies = []
        for ci, col in enumerate((c0, c1, c2, c3)):
            for j in range(n_ch):
                copies.append(pltpu.async_copy(
                    col.at[nd_v.at[j]],
                    colbuf.at[ci, pl.ds(j * CH, CH)],
                    sem,
                ))
        for c in copies:
            c.wait()

        lane = lax.iota(jnp.int32, _L)
        bid0 = lane >> 2
        col0 = lane & 3
        for v in range(b_per_w * D // _L):
            bid = bid0 + (v * _L // D)
            vals = plsc.load_gather(colbuf, [col0, bid])
            plsc.store_scatter(out_v, [bid, col0], vals)

        pltpu.sync_copy(out_v, out_hbm.at[pl.ds(base, b_per_w)])

    node2d = node.reshape(B // CH, CH)
    return k(node2d, *cols)


def kernel(node, table):
    return _gather(node.astype(jnp.int32), table)


# speedup vs baseline: 6.7470x; 1.4261x over previous
import functools

import jax
import jax.numpy as jnp
from jax import lax
from jax.experimental import pallas as pl
from jax.experimental.pallas import tpu as pltpu
from jax.experimental.pallas import tpu_sc as plsc


@jax.jit
def _gather(node, table):
    B, = node.shape
    V, D = table.shape
    cols = lax.optimization_barrier(tuple(table[:, c] for c in range(D)))
    info = plsc.get_sparse_core_info()
    NC, NS = info.num_cores, info.num_subcores
    NW = NC * NS
    b_per_w = B // NW
    CH = 128
    n_ch = b_per_w // CH
    mesh = plsc.VectorSubcoreMesh(core_axis_name="c", subcore_axis_name="s")

    @functools.partial(
        pl.kernel,
        mesh=mesh,
        out_type=jax.ShapeDtypeStruct((D, B), jnp.float32),
        scratch_types=[
            pltpu.VMEM((n_ch, CH), jnp.int32),
            pltpu.VMEM((D, b_per_w), jnp.float32),
            pltpu.SemaphoreType.DMA,
        ],
        compiler_params=pltpu.CompilerParams(needs_layout_passes=False),
    )
    def k(node_hbm, c0, c1, c2, c3, out_hbm, nd_v, colbuf, sem):
        wid = lax.axis_index("s") * NC + lax.axis_index("c")
        base = wid * b_per_w
        pltpu.sync_copy(node_hbm.at[pl.ds(wid * n_ch, n_ch)], nd_v)

        copies = []
        for ci, col in enumerate((c0, c1, c2, c3)):
            for j in range(n_ch):
                copies.append(pltpu.async_copy(
                    col.at[nd_v.at[j]],
                    colbuf.at[ci, pl.ds(j * CH, CH)],
                    sem,
                ))
        for c in copies:
            c.wait()

        for ci in range(D):
            pltpu.sync_copy(
                colbuf.at[ci],
                out_hbm.at[ci, pl.ds(base, b_per_w)],
            )

    node2d = node.reshape(B // CH, CH)
    return k(node2d, *cols).T


def kernel(node, table):
    return _gather(node.astype(jnp.int32), table)
